# baseline (speedup 1.0000x reference)
import jax
import jax.numpy as jnp
from jax.experimental import pallas as pl
from jax.experimental.pallas import tpu as pltpu

_B = 8
_N = 512
_D = 512
_DA = _D + 8
_M = 16384
_K = 9
_T = 8
_BT = _B * _T

_TR = 512
_TC = 2048
_NR = (_B * _N) // _TR
_NC = _M // _TC
_IPT = _TR // _N


def _build_body(y_ref, aug_ref, yn_ref):
    y32 = y_ref[...]
    ynorm = jnp.sum(y32 * y32, axis=0, keepdims=True)
    yn_ref[...] = ynorm
    hi = ynorm.astype(jnp.bfloat16)
    lo = (ynorm - hi.astype(jnp.float32)).astype(jnp.bfloat16)
    aug_ref[0:_D, :] = -2.0 * y32.astype(jnp.bfloat16)
    sub = jax.lax.broadcasted_iota(jnp.int32, (8, _TC), 0)
    tail = jnp.where(sub == 0, jnp.broadcast_to(hi, (8, _TC)),
                     jnp.where(sub == 1, jnp.broadcast_to(lo, (8, _TC)),
                               jnp.float32(0.0))).astype(jnp.bfloat16)
    aug_ref[_D:_DA, :] = tail


def _stage1_body(xa_ref, x32_ref, y_ref, scores_ref, cand_ref, qc_ref,
                 minval):
    i = pl.program_id(0)
    j = pl.program_id(1)

    @pl.when(j == 0)
    def _init():
        minval[...] = jnp.full_like(minval[...], jnp.inf)

    s = jax.lax.dot_general(xa_ref[...], y_ref[...], (((1,), (0,)), ((), ())),
                            preferred_element_type=jnp.float32)
    tmin = jnp.min(s, axis=1, keepdims=True)
    minval[...] = jnp.minimum(minval[...], tmin)

    @pl.when(j == _NC - 1)
    def _finalize():
        x32 = x32_ref[...]
        xnorm = jnp.sum(x32 * x32, axis=1, keepdims=True)
        sc = jnp.sqrt(jnp.clip(xnorm + minval[...], 1e-12, None))
        scores_ref[...] = sc
        rowio = jax.lax.broadcasted_iota(jnp.int32, (_N, 1), 0)
        for bl in range(_IPT):
            b = i * _IPT + bl
            seg = sc[bl * _N:(bl + 1) * _N, :]
            xseg = x32[bl * _N:(bl + 1) * _N, :]
            for t in range(_T):
                m = jnp.argmax(seg[:, 0], axis=0).astype(jnp.int32)
                sel = rowio == m
                cand_ref[pl.ds(b * _T + t, 1), :] = (
                    jnp.full((1, 1), 0, jnp.int32) + b * _N + m)
                qc_ref[pl.ds(b * _T + t, 1), :] = jnp.sum(
                    jnp.where(sel, xseg, 0.0), axis=0, keepdims=True)
                seg = jnp.where(sel, -jnp.inf, seg)


def _rescore_body(qc_ref, cr_ref, y_ref, yn_ref,
                  nnidx_ref, maxsc_ref, dq_ref,
                  minv, mina, dsq):
    j = pl.program_id(0)

    @pl.when(j == 0)
    def _init():
        minv[...] = jnp.full_like(minv[...], jnp.inf)
        mina[...] = jnp.zeros_like(mina[...])

    qc = qc_ref[...]
    prod = jax.lax.dot_general(qc, y_ref[...], (((1,), (0,)), ((), ())),
                               preferred_element_type=jnp.float32)
    s = yn_ref[...] - 2.0 * prod
    dsq[:, pl.ds(j * _TC, _TC)] = s
    tmin = jnp.min(s, axis=1, keepdims=True)
    targ = jnp.argmin(s, axis=1).astype(jnp.int32)[:, None] + j * _TC
    better = tmin < minv[...]
    mina[...] = jnp.where(better, targ, mina[...])
    minv[...] = jnp.where(better, tmin, minv[...])

    @pl.when(j == _NC - 1)
    def _finalize():
        qnorm = jnp.sum(qc * qc, axis=1, keepdims=True)
        sc = jnp.sqrt(jnp.clip(qnorm + minv[...], 1e-12, None))
        rows = cr_ref[...]
        for b in range(_B):
            sl = slice(b * _T, (b + 1) * _T)
            seg = sc[sl, :]
            mx = jnp.max(seg)
            hit = seg == mx
            rseg = rows[sl, :]
            rsel = jnp.min(jnp.where(hit, rseg, _B * _N))
            selc = jnp.logical_and(hit, rseg == rsel)
            nnidx_ref[b:b + 1, :] = jnp.sum(
                jnp.where(selc, mina[sl, :], 0), axis=0, keepdims=True)
            maxsc_ref[b:b + 1, :] = jnp.full((1, 1), 0.0) + mx
            qn_sel = jnp.sum(jnp.where(selc, qnorm[sl, :], 0.0))
            dq_ref[b:b + 1, :] = qn_sel + jnp.sum(
                jnp.where(selc, dsq[sl, :], 0.0), axis=0, keepdims=True)


def _gather_body(idx_ref, bank_ref, out_ref):
    out_ref[...] = bank_ref[...]


def _support_body(nn_ref, dq_ref, maxsc_ref, y_ref, yn_ref, pred_ref, dn_sq):
    j = pl.program_id(0)
    nn = nn_ref[...]
    prod = jax.lax.dot_general(nn, y_ref[...], (((1,), (0,)), ((), ())),
                               preferred_element_type=jnp.float32)
    nnorm = jnp.sum(nn * nn, axis=1, keepdims=True)
    dn_sq[:, pl.ds(j * _TC, _TC)] = nnorm + (yn_ref[...] - 2.0 * prod)

    @pl.when(j == _NC - 1)
    def _finalize():
        dq = dq_ref[...]
        dn = dn_sq[...]
        colio = jax.lax.broadcasted_iota(jnp.int32, (_B, _M), 1)
        lane16 = jax.lax.broadcasted_iota(jnp.int32, (_B, 16), 1)
        dm = jnp.full((_B, 16), -jnp.inf, dtype=jnp.float32)
        for k in range(_K):
            midx = jnp.argmin(dn, axis=1).astype(jnp.int32)[:, None]
            onehot = colio == midx
            dq_k = jnp.sum(jnp.where(onehot, dq, 0.0), axis=1,
                           keepdims=True)
            dist_k = jnp.sqrt(jnp.clip(dq_k, 1e-12, None))
            dm = jnp.where(lane16 == k, jnp.broadcast_to(dist_k, (_B, 16)),
                           dm)
            dn = jnp.where(onehot, jnp.inf, dn)
        mx = jnp.max(dm, axis=1, keepdims=True)
        e = jnp.exp(dm - mx)
        w0 = e[:, 0:1] / jnp.sum(e, axis=1, keepdims=True)
        pred_ref[...] = (1.0 - w0) * maxsc_ref[...]


def _augment(v):
    bf16 = jnp.bfloat16
    n = v.shape[0]
    return jnp.concatenate(
        [v.astype(bf16), jnp.ones((n, 2), bf16), jnp.zeros((n, 6), bf16)],
        axis=1)


@jax.jit
def kernel(embedding, memory_bank):
    f32, i32 = jnp.float32, jnp.int32
    bank_t = memory_bank.T
    x_aug = _augment(embedding)

    y_aug, yn32 = pl.pallas_call(
        _build_body,
        grid=(_NC,),
        in_specs=[pl.BlockSpec((_D, _TC), lambda j: (0, j))],
        out_specs=[
            pl.BlockSpec((_DA, _TC), lambda j: (0, j)),
            pl.BlockSpec((1, _TC), lambda j: (0, j)),
        ],
        out_shape=[
            jax.ShapeDtypeStruct((_DA, _M), jnp.bfloat16),
            jax.ShapeDtypeStruct((1, _M), f32),
        ],
        compiler_params=pltpu.CompilerParams(
            dimension_semantics=("arbitrary",)),
    )(bank_t)

    scores, cand, q_cand = pl.pallas_call(
        _stage1_body,
        grid=(_NR, _NC),
        in_specs=[
            pl.BlockSpec((_TR, _DA), lambda i, j: (i, 0)),
            pl.BlockSpec((_TR, _D), lambda i, j: (i, 0)),
            pl.BlockSpec((_DA, _TC), lambda i, j: (0, j)),
        ],
        out_specs=[
            pl.BlockSpec((_TR, 1), lambda i, j: (i, 0)),
            pl.BlockSpec((_BT, 1), lambda i, j: (0, 0)),
            pl.BlockSpec((_BT, _D), lambda i, j: (0, 0)),
        ],
        out_shape=[
            jax.ShapeDtypeStruct((_B * _N, 1), f32),
            jax.ShapeDtypeStruct((_BT, 1), i32),
            jax.ShapeDtypeStruct((_BT, _D), f32),
        ],
        scratch_shapes=[pltpu.VMEM((_TR, 1), f32)],
        compiler_params=pltpu.CompilerParams(
            dimension_semantics=("arbitrary", "arbitrary")),
    )(x_aug, embedding, y_aug)

    nnidx, maxsc, dq = pl.pallas_call(
        _rescore_body,
        grid=(_NC,),
        in_specs=[
            pl.BlockSpec((_BT, _D), lambda j: (0, 0)),
            pl.BlockSpec((_BT, 1), lambda j: (0, 0)),
            pl.BlockSpec((_D, _TC), lambda j: (0, j)),
            pl.BlockSpec((1, _TC), lambda j: (0, j)),
        ],
        out_specs=[
            pl.BlockSpec((_B, 1), lambda j: (0, 0)),
            pl.BlockSpec((_B, 1), lambda j: (0, 0)),
            pl.BlockSpec((_B, _M), lambda j: (0, 0)),
        ],
        out_shape=[
            jax.ShapeDtypeStruct((_B, 1), i32),
            jax.ShapeDtypeStruct((_B, 1), f32),
            jax.ShapeDtypeStruct((_B, _M), f32),
        ],
        scratch_shapes=[
            pltpu.VMEM((_BT, 1), f32),
            pltpu.VMEM((_BT, 1), i32),
            pltpu.VMEM((_BT, _M), f32),
        ],
        compiler_params=pltpu.CompilerParams(
            dimension_semantics=("arbitrary",)),
    )(q_cand, cand, bank_t, yn32)

    nn8 = pl.pallas_call(
        _gather_body,
        grid_spec=pltpu.PrefetchScalarGridSpec(
            num_scalar_prefetch=1,
            grid=(_B,),
            in_specs=[pl.BlockSpec((1, 1, _D), lambda b, idx: (idx[b], 0, 0))],
            out_specs=pl.BlockSpec((1, 1, _D), lambda b, idx: (b, 0, 0)),
        ),
        out_shape=jax.ShapeDtypeStruct((_B, 1, _D), f32),
    )(nnidx.reshape(_B), memory_bank.reshape(_M, 1, _D)).reshape(_B, _D)

    pred = pl.pallas_call(
        _support_body,
        grid=(_NC,),
        in_specs=[
            pl.BlockSpec((_B, _D), lambda j: (0, 0)),
            pl.BlockSpec((_B, _M), lambda j: (0, 0)),
            pl.BlockSpec((_B, 1), lambda j: (0, 0)),
            pl.BlockSpec((_D, _TC), lambda j: (0, j)),
            pl.BlockSpec((1, _TC), lambda j: (0, j)),
        ],
        out_specs=pl.BlockSpec((_B, 1), lambda j: (0, 0)),
        out_shape=jax.ShapeDtypeStruct((_B, 1), f32),
        scratch_shapes=[pltpu.VMEM((_B, _M), f32)],
        compiler_params=pltpu.CompilerParams(
            dimension_semantics=("arbitrary",)),
    )(nn8, dq, maxsc, bank_t, yn32)

    return scores.reshape(_B, _N), pred.reshape(_B)


# speedup vs baseline: 1.0366x; 1.0366x over previous
import jax
import jax.numpy as jnp
from jax.experimental import pallas as pl
from jax.experimental.pallas import tpu as pltpu

_B = 8
_N = 512
_D = 512
_DA = _D + 8
_M = 16384
_K = 9
_T = 4
_BT = _B * _T

_TR = 512
_TC = 2048
_NR = (_B * _N) // _TR
_NC = _M // _TC
_IPT = _TR // _N


def _build_body(y_ref, aug_ref, yn_ref):
    y32 = y_ref[...]
    ynorm = jnp.sum(y32 * y32, axis=0, keepdims=True)
    yn_ref[...] = ynorm
    hi = ynorm.astype(jnp.bfloat16)
    lo = (ynorm - hi.astype(jnp.float32)).astype(jnp.bfloat16)
    aug_ref[0:_D, :] = -2.0 * y32.astype(jnp.bfloat16)
    sub = jax.lax.broadcasted_iota(jnp.int32, (8, _TC), 0)
    tail = jnp.where(sub == 0, jnp.broadcast_to(hi, (8, _TC)),
                     jnp.where(sub == 1, jnp.broadcast_to(lo, (8, _TC)),
                               jnp.float32(0.0))).astype(jnp.bfloat16)
    aug_ref[_D:_DA, :] = tail


def _stage1_body(xa_ref, x32_ref, y_ref, scores_ref, cand_ref, qc_ref,
                 minval):
    i = pl.program_id(0)
    j = pl.program_id(1)

    @pl.when(j == 0)
    def _init():
        minval[...] = jnp.full_like(minval[...], jnp.inf)

    xa = xa_ref[...]
    tmin = None
    for c in range(_TC // 512):
        sc_ = jax.lax.dot_general(xa, y_ref[:, c * 512:(c + 1) * 512],
                                  (((1,), (0,)), ((), ())),
                                  preferred_element_type=jnp.float32)
        cm = jnp.min(sc_, axis=1, keepdims=True)
        tmin = cm if tmin is None else jnp.minimum(tmin, cm)
    minval[...] = jnp.minimum(minval[...], tmin)

    @pl.when(j == _NC - 1)
    def _finalize():
        x32 = x32_ref[...]
        xnorm = jnp.sum(x32 * x32, axis=1, keepdims=True)
        sc = jnp.sqrt(jnp.clip(xnorm + minval[...], 1e-12, None))
        scores_ref[...] = sc
        rowio = jax.lax.broadcasted_iota(jnp.int32, (_N, 1), 0)
        for bl in range(_IPT):
            b = i * _IPT + bl
            seg = sc[bl * _N:(bl + 1) * _N, :]
            xseg = x32[bl * _N:(bl + 1) * _N, :]
            for t in range(_T):
                m = jnp.argmax(seg[:, 0], axis=0).astype(jnp.int32)
                sel = rowio == m
                cand_ref[pl.ds(b * _T + t, 1), :] = (
                    jnp.full((1, 1), 0, jnp.int32) + b * _N + m)
                qc_ref[pl.ds(b * _T + t, 1), :] = jnp.sum(
                    jnp.where(sel, xseg, 0.0), axis=0, keepdims=True)
                seg = jnp.where(sel, -jnp.inf, seg)


def _rescore_body(qc_ref, cr_ref, y_ref, yn_ref,
                  nnidx_ref, maxsc_ref, dq_ref,
                  minv, mina, dsq):
    j = pl.program_id(0)

    @pl.when(j == 0)
    def _init():
        minv[...] = jnp.full_like(minv[...], jnp.inf)
        mina[...] = jnp.zeros_like(mina[...])

    qc = qc_ref[...]
    prod = jax.lax.dot_general(qc, y_ref[...], (((1,), (0,)), ((), ())),
                               preferred_element_type=jnp.float32)
    s = yn_ref[...] - 2.0 * prod
    dsq[:, pl.ds(j * _TC, _TC)] = s
    tmin = jnp.min(s, axis=1, keepdims=True)
    targ = jnp.argmin(s, axis=1).astype(jnp.int32)[:, None] + j * _TC
    better = tmin < minv[...]
    mina[...] = jnp.where(better, targ, mina[...])
    minv[...] = jnp.where(better, tmin, minv[...])

    @pl.when(j == _NC - 1)
    def _finalize():
        qnorm = jnp.sum(qc * qc, axis=1, keepdims=True)
        sc = jnp.sqrt(jnp.clip(qnorm + minv[...], 1e-12, None))
        rows = cr_ref[...]
        for b in range(_B):
            sl = slice(b * _T, (b + 1) * _T)
            seg = sc[sl, :]
            mx = jnp.max(seg)
            hit = seg == mx
            rseg = rows[sl, :]
            rsel = jnp.min(jnp.where(hit, rseg, _B * _N))
            selc = jnp.logical_and(hit, rseg == rsel)
            nnidx_ref[b:b + 1, :] = jnp.sum(
                jnp.where(selc, mina[sl, :], 0), axis=0, keepdims=True)
            maxsc_ref[b:b + 1, :] = jnp.full((1, 1), 0.0) + mx
            qn_sel = jnp.sum(jnp.where(selc, qnorm[sl, :], 0.0))
            dq_ref[b:b + 1, :] = qn_sel + jnp.sum(
                jnp.where(selc, dsq[sl, :], 0.0), axis=0, keepdims=True)


def _gather_body(idx_ref, bank_ref, out_ref):
    out_ref[...] = bank_ref[...]


def _support_body(nn_ref, dq_ref, maxsc_ref, y_ref, yn_ref, pred_ref, dn_sq):
    j = pl.program_id(0)
    nn = nn_ref[...]
    prod = jax.lax.dot_general(nn, y_ref[...], (((1,), (0,)), ((), ())),
                               preferred_element_type=jnp.float32)
    nnorm = jnp.sum(nn * nn, axis=1, keepdims=True)
    dn_sq[:, pl.ds(j * _TC, _TC)] = nnorm + (yn_ref[...] - 2.0 * prod)

    @pl.when(j == _NC - 1)
    def _finalize():
        dq = dq_ref[...]
        dn = dn_sq[...]
        colio = jax.lax.broadcasted_iota(jnp.int32, (_B, _M), 1)
        lane16 = jax.lax.broadcasted_iota(jnp.int32, (_B, 16), 1)
        dm = jnp.full((_B, 16), -jnp.inf, dtype=jnp.float32)
        for k in range(_K):
            midx = jnp.argmin(dn, axis=1).astype(jnp.int32)[:, None]
            onehot = colio == midx
            dq_k = jnp.sum(jnp.where(onehot, dq, 0.0), axis=1,
                           keepdims=True)
            dist_k = jnp.sqrt(jnp.clip(dq_k, 1e-12, None))
            dm = jnp.where(lane16 == k, jnp.broadcast_to(dist_k, (_B, 16)),
                           dm)
            dn = jnp.where(onehot, jnp.inf, dn)
        mx = jnp.max(dm, axis=1, keepdims=True)
        e = jnp.exp(dm - mx)
        w0 = e[:, 0:1] / jnp.sum(e, axis=1, keepdims=True)
        pred_ref[...] = (1.0 - w0) * maxsc_ref[...]


def _augment(v):
    bf16 = jnp.bfloat16
    n = v.shape[0]
    return jnp.concatenate(
        [v.astype(bf16), jnp.ones((n, 2), bf16), jnp.zeros((n, 6), bf16)],
        axis=1)


@jax.jit
def kernel(embedding, memory_bank):
    f32, i32 = jnp.float32, jnp.int32
    bank_t = memory_bank.T
    x_aug = _augment(embedding)

    y_aug, yn32 = pl.pallas_call(
        _build_body,
        grid=(_NC,),
        in_specs=[pl.BlockSpec((_D, _TC), lambda j: (0, j))],
        out_specs=[
            pl.BlockSpec((_DA, _TC), lambda j: (0, j)),
            pl.BlockSpec((1, _TC), lambda j: (0, j)),
        ],
        out_shape=[
            jax.ShapeDtypeStruct((_DA, _M), jnp.bfloat16),
            jax.ShapeDtypeStruct((1, _M), f32),
        ],
        compiler_params=pltpu.CompilerParams(
            dimension_semantics=("arbitrary",)),
    )(bank_t)

    scores, cand, q_cand = pl.pallas_call(
        _stage1_body,
        grid=(_NR, _NC),
        in_specs=[
            pl.BlockSpec((_TR, _DA), lambda i, j: (i, 0)),
            pl.BlockSpec((_TR, _D), lambda i, j: (i, 0)),
            pl.BlockSpec((_DA, _TC), lambda i, j: (0, j)),
        ],
        out_specs=[
            pl.BlockSpec((_TR, 1), lambda i, j: (i, 0)),
            pl.BlockSpec((_BT, 1), lambda i, j: (0, 0)),
            pl.BlockSpec((_BT, _D), lambda i, j: (0, 0)),
        ],
        out_shape=[
            jax.ShapeDtypeStruct((_B * _N, 1), f32),
            jax.ShapeDtypeStruct((_BT, 1), i32),
            jax.ShapeDtypeStruct((_BT, _D), f32),
        ],
        scratch_shapes=[pltpu.VMEM((_TR, 1), f32)],
        compiler_params=pltpu.CompilerParams(
            dimension_semantics=("arbitrary", "arbitrary")),
    )(x_aug, embedding, y_aug)

    nnidx, maxsc, dq = pl.pallas_call(
        _rescore_body,
        grid=(_NC,),
        in_specs=[
            pl.BlockSpec((_BT, _D), lambda j: (0, 0)),
            pl.BlockSpec((_BT, 1), lambda j: (0, 0)),
            pl.BlockSpec((_D, _TC), lambda j: (0, j)),
            pl.BlockSpec((1, _TC), lambda j: (0, j)),
        ],
        out_specs=[
            pl.BlockSpec((_B, 1), lambda j: (0, 0)),
            pl.BlockSpec((_B, 1), lambda j: (0, 0)),
            pl.BlockSpec((_B, _M), lambda j: (0, 0)),
        ],
        out_shape=[
            jax.ShapeDtypeStruct((_B, 1), i32),
            jax.ShapeDtypeStruct((_B, 1), f32),
            jax.ShapeDtypeStruct((_B, _M), f32),
        ],
        scratch_shapes=[
            pltpu.VMEM((_BT, 1), f32),
            pltpu.VMEM((_BT, 1), i32),
            pltpu.VMEM((_BT, _M), f32),
        ],
        compiler_params=pltpu.CompilerParams(
            dimension_semantics=("arbitrary",)),
    )(q_cand, cand, bank_t, yn32)

    nn8 = pl.pallas_call(
        _gather_body,
        grid_spec=pltpu.PrefetchScalarGridSpec(
            num_scalar_prefetch=1,
            grid=(_B,),
            in_specs=[pl.BlockSpec((1, 1, _D), lambda b, idx: (idx[b], 0, 0))],
            out_specs=pl.BlockSpec((1, 1, _D), lambda b, idx: (b, 0, 0)),
        ),
        out_shape=jax.ShapeDtypeStruct((_B, 1, _D), f32),
    )(nnidx.reshape(_B), memory_bank.reshape(_M, 1, _D)).reshape(_B, _D)

    pred = pl.pallas_call(
        _support_body,
        grid=(_NC,),
        in_specs=[
            pl.BlockSpec((_B, _D), lambda j: (0, 0)),
            pl.BlockSpec((_B, _M), lambda j: (0, 0)),
            pl.BlockSpec((_B, 1), lambda j: (0, 0)),
            pl.BlockSpec((_D, _TC), lambda j: (0, j)),
            pl.BlockSpec((1, _TC), lambda j: (0, j)),
        ],
        out_specs=pl.BlockSpec((_B, 1), lambda j: (0, 0)),
        out_shape=jax.ShapeDtypeStruct((_B, 1), f32),
        scratch_shapes=[pltpu.VMEM((_B, _M), f32)],
        compiler_params=pltpu.CompilerParams(
            dimension_semantics=("arbitrary",)),
    )(nn8, dq, maxsc, bank_t, yn32)

    return scores.reshape(_B, _N), pred.reshape(_B)


# speedup vs baseline: 1.2140x; 1.1711x over previous
import jax
import jax.numpy as jnp
from jax.experimental import pallas as pl
from jax.experimental.pallas import tpu as pltpu

_B = 8
_N = 512
_D = 512
_M = 16384
_K = 9

_TR = 512
_TC = 2048
_NR = (_B * _N) // _TR
_NC = _M // _TC


def _stage1_body(x_ref, y_ref, scores_ref, q_ref, maxsc_ref, minval, yn_c):
    i = pl.program_id(0)
    j = pl.program_id(1)

    @pl.when(j == 0)
    def _init():
        minval[...] = jnp.full_like(minval[...], jnp.inf)

    x = x_ref[...]
    prod = jax.lax.dot_general(x, y_ref[...], (((1,), (0,)), ((), ())),
                               preferred_element_type=jnp.float32)
    cols = pl.ds(j * _TC, _TC)

    @pl.when(i == 0)
    def _norms():
        y32 = y_ref[...]
        yn_c[:, cols] = jnp.sum(y32 * y32, axis=0, keepdims=True)

    s = yn_c[:, cols] - 2.0 * prod
    tmin = jnp.min(s, axis=1, keepdims=True)
    minval[...] = jnp.minimum(minval[...], tmin)

    @pl.when(j == _NC - 1)
    def _finalize():
        xnorm = jnp.sum(x * x, axis=1, keepdims=True)
        sc = jnp.sqrt(jnp.clip(xnorm + minval[...], 1e-12, None))
        scores_ref[...] = sc
        rowio = jax.lax.broadcasted_iota(jnp.int32, (_TR, 1), 0)
        m = jnp.argmax(sc[:, 0], axis=0).astype(jnp.int32)
        sel = rowio == m
        q_ref[pl.ds(i, 1), :] = jnp.sum(jnp.where(sel, x, 0.0), axis=0,
                                        keepdims=True)
        maxsc_ref[pl.ds(i, 1), :] = jnp.sum(
            jnp.where(sel, sc, 0.0), axis=0, keepdims=True)


def _stage2a_body(q_ref, y_ref, dq_ref, nnidx_ref, minv, mina):
    j = pl.program_id(0)

    @pl.when(j == 0)
    def _init():
        minv[...] = jnp.full_like(minv[...], jnp.inf)
        mina[...] = jnp.zeros_like(mina[...])

    q = q_ref[...]
    prod = jax.lax.dot_general(q, y_ref[...], (((1,), (0,)), ((), ())),
                               preferred_element_type=jnp.float32)
    y32 = y_ref[...]
    ynorm = jnp.sum(y32 * y32, axis=0, keepdims=True)
    qnorm = jnp.sum(q * q, axis=1, keepdims=True)
    d = qnorm + (ynorm - 2.0 * prod)
    dq_ref[:, pl.ds(j * _TC, _TC)] = d
    tmin = jnp.min(d, axis=1, keepdims=True)
    targ = jnp.argmin(d, axis=1).astype(jnp.int32)[:, None] + j * _TC
    better = tmin < minv[...]
    mina[...] = jnp.where(better, targ, mina[...])
    minv[...] = jnp.where(better, tmin, minv[...])

    @pl.when(j == _NC - 1)
    def _finalize():
        nnidx_ref[...] = mina[...]


def _gather_body(idx_ref, bank_ref, out_ref):
    out_ref[...] = bank_ref[...]


def _stage2b_body(nn_ref, dq_ref, maxsc_ref, y_ref, pred_ref, dn_sq):
    j = pl.program_id(0)
    nn = nn_ref[...]
    prod = jax.lax.dot_general(nn, y_ref[...], (((1,), (0,)), ((), ())),
                               preferred_element_type=jnp.float32)
    y32 = y_ref[...]
    ynorm = jnp.sum(y32 * y32, axis=0, keepdims=True)
    nnorm = jnp.sum(nn * nn, axis=1, keepdims=True)
    dn_sq[:, pl.ds(j * _TC, _TC)] = nnorm + (ynorm - 2.0 * prod)

    @pl.when(j == _NC - 1)
    def _finalize():
        dq = dq_ref[...]
        dn = dn_sq[...]
        colio = jax.lax.broadcasted_iota(jnp.int32, (_B, _M), 1)
        lane16 = jax.lax.broadcasted_iota(jnp.int32, (_B, 16), 1)
        dm = jnp.full((_B, 16), -jnp.inf, dtype=jnp.float32)
        for k in range(_K):
            midx = jnp.argmin(dn, axis=1).astype(jnp.int32)[:, None]
            onehot = colio == midx
            dq_k = jnp.sum(jnp.where(onehot, dq, 0.0), axis=1,
                           keepdims=True)
            dist_k = jnp.sqrt(jnp.clip(dq_k, 1e-12, None))
            dm = jnp.where(lane16 == k, jnp.broadcast_to(dist_k, (_B, 16)),
                           dm)
            dn = jnp.where(onehot, jnp.inf, dn)
        mx = jnp.max(dm, axis=1, keepdims=True)
        e = jnp.exp(dm - mx)
        w0 = e[:, 0:1] / jnp.sum(e, axis=1, keepdims=True)
        pred_ref[...] = (1.0 - w0) * maxsc_ref[...]


@jax.jit
def kernel(embedding, memory_bank):
    f32, i32 = jnp.float32, jnp.int32
    bank_t = memory_bank.T

    scores, q8, maxsc = pl.pallas_call(
        _stage1_body,
        grid=(_NR, _NC),
        in_specs=[
            pl.BlockSpec((_TR, _D), lambda i, j: (i, 0)),
            pl.BlockSpec((_D, _TC), lambda i, j: (0, j)),
        ],
        out_specs=[
            pl.BlockSpec((_TR, 1), lambda i, j: (i, 0)),
            pl.BlockSpec((_B, _D), lambda i, j: (0, 0)),
            pl.BlockSpec((_B, 1), lambda i, j: (0, 0)),
        ],
        out_shape=[
            jax.ShapeDtypeStruct((_B * _N, 1), f32),
            jax.ShapeDtypeStruct((_B, _D), f32),
            jax.ShapeDtypeStruct((_B, 1), f32),
        ],
        scratch_shapes=[
            pltpu.VMEM((_TR, 1), f32),
            pltpu.VMEM((1, _M), f32),
        ],
        compiler_params=pltpu.CompilerParams(
            dimension_semantics=("arbitrary", "arbitrary")),
    )(embedding, bank_t)

    dq, nnidx = pl.pallas_call(
        _stage2a_body,
        grid=(_NC,),
        in_specs=[
            pl.BlockSpec((_B, _D), lambda j: (0, 0)),
            pl.BlockSpec((_D, _TC), lambda j: (0, j)),
        ],
        out_specs=[
            pl.BlockSpec((_B, _M), lambda j: (0, 0)),
            pl.BlockSpec((_B, 1), lambda j: (0, 0)),
        ],
        out_shape=[
            jax.ShapeDtypeStruct((_B, _M), f32),
            jax.ShapeDtypeStruct((_B, 1), i32),
        ],
        scratch_shapes=[
            pltpu.VMEM((_B, 1), f32),
            pltpu.VMEM((_B, 1), i32),
        ],
        compiler_params=pltpu.CompilerParams(
            dimension_semantics=("arbitrary",)),
    )(q8, bank_t)

    nn8 = pl.pallas_call(
        _gather_body,
        grid_spec=pltpu.PrefetchScalarGridSpec(
            num_scalar_prefetch=1,
            grid=(_B,),
            in_specs=[pl.BlockSpec((1, 1, _D), lambda b, idx: (idx[b], 0, 0))],
            out_specs=pl.BlockSpec((1, 1, _D), lambda b, idx: (b, 0, 0)),
        ),
        out_shape=jax.ShapeDtypeStruct((_B, 1, _D), f32),
    )(nnidx.reshape(_B), memory_bank.reshape(_M, 1, _D)).reshape(_B, _D)

    pred = pl.pallas_call(
        _stage2b_body,
        grid=(_NC,),
        in_specs=[
            pl.BlockSpec((_B, _D), lambda j: (0, 0)),
            pl.BlockSpec((_B, _M), lambda j: (0, 0)),
            pl.BlockSpec((_B, 1), lambda j: (0, 0)),
            pl.BlockSpec((_D, _TC), lambda j: (0, j)),
        ],
        out_specs=pl.BlockSpec((_B, 1), lambda j: (0, 0)),
        out_shape=jax.ShapeDtypeStruct((_B, 1), f32),
        scratch_shapes=[pltpu.VMEM((_B, _M), f32)],
        compiler_params=pltpu.CompilerParams(
            dimension_semantics=("arbitrary",)),
    )(nn8, dq, maxsc, bank_t)

    return scores.reshape(_B, _N), pred.reshape(_B)


# speedup vs baseline: 1.2851x; 1.0586x over previous
import jax
import jax.numpy as jnp
from jax.experimental import pallas as pl
from jax.experimental.pallas import tpu as pltpu

_B = 8
_N = 512
_D = 512
_M = 16384
_K = 9

_TR = 512
_TC = 2048
_NR = (_B * _N) // _TR
_NC = _M // _TC


def _stage1_body(x_ref, y_ref, scores_ref, q_ref, maxsc_ref, minval):
    i = pl.program_id(0)
    j = pl.program_id(1)

    @pl.when(j == 0)
    def _init():
        minval[...] = jnp.full_like(minval[...], jnp.inf)

    x = x_ref[...]
    y = y_ref[...]
    prod = jax.lax.dot_general(x, y, (((1,), (0,)), ((), ())),
                               preferred_element_type=jnp.float32)
    ynorm = jnp.sum(y * y, axis=0, keepdims=True)
    s = ynorm - 2.0 * prod
    tmin = jnp.min(s, axis=1, keepdims=True)
    minval[...] = jnp.minimum(minval[...], tmin)

    @pl.when(j == _NC - 1)
    def _finalize():
        xnorm = jnp.sum(x * x, axis=1, keepdims=True)
        sc = jnp.sqrt(jnp.clip(xnorm + minval[...], 1e-12, None))
        scores_ref[...] = sc
        rowio = jax.lax.broadcasted_iota(jnp.int32, (_TR, 1), 0)
        m = jnp.argmax(sc[:, 0], axis=0).astype(jnp.int32)
        sel = rowio == m
        q_ref[pl.ds(i, 1), :] = jnp.sum(jnp.where(sel, x, 0.0), axis=0,
                                        keepdims=True)
        maxsc_ref[pl.ds(i, 1), :] = jnp.sum(
            jnp.where(sel, sc, 0.0), axis=0, keepdims=True)


def _stage2a_body(q_ref, y_ref, dq_ref, nnidx_ref, minv, mina):
    j = pl.program_id(0)

    @pl.when(j == 0)
    def _init():
        minv[...] = jnp.full_like(minv[...], jnp.inf)
        mina[...] = jnp.zeros_like(mina[...])

    q = q_ref[...]
    prod = jax.lax.dot_general(q, y_ref[...], (((1,), (0,)), ((), ())),
                               preferred_element_type=jnp.float32)
    y32 = y_ref[...]
    ynorm = jnp.sum(y32 * y32, axis=0, keepdims=True)
    qnorm = jnp.sum(q * q, axis=1, keepdims=True)
    d = qnorm + (ynorm - 2.0 * prod)
    dq_ref[:, pl.ds(j * _TC, _TC)] = d
    tmin = jnp.min(d, axis=1, keepdims=True)
    targ = jnp.argmin(d, axis=1).astype(jnp.int32)[:, None] + j * _TC
    better = tmin < minv[...]
    mina[...] = jnp.where(better, targ, mina[...])
    minv[...] = jnp.where(better, tmin, minv[...])

    @pl.when(j == _NC - 1)
    def _finalize():
        nnidx_ref[...] = mina[...]


def _gather_body(idx_ref, bank_ref, out_ref):
    out_ref[...] = bank_ref[...]


def _stage2b_body(nn_ref, dq_ref, maxsc_ref, y_ref, pred_ref, dn_sq):
    j = pl.program_id(0)
    nn = nn_ref[...]
    prod = jax.lax.dot_general(nn, y_ref[...], (((1,), (0,)), ((), ())),
                               preferred_element_type=jnp.float32)
    y32 = y_ref[...]
    ynorm = jnp.sum(y32 * y32, axis=0, keepdims=True)
    nnorm = jnp.sum(nn * nn, axis=1, keepdims=True)
    dn_sq[:, pl.ds(j * _TC, _TC)] = nnorm + (ynorm - 2.0 * prod)

    @pl.when(j == _NC - 1)
    def _finalize():
        dq = dq_ref[...]
        dn = dn_sq[...]
        colio = jax.lax.broadcasted_iota(jnp.int32, (_B, _M), 1)
        lane16 = jax.lax.broadcasted_iota(jnp.int32, (_B, 16), 1)
        dm = jnp.full((_B, 16), -jnp.inf, dtype=jnp.float32)
        for k in range(_K):
            midx = jnp.argmin(dn, axis=1).astype(jnp.int32)[:, None]
            onehot = colio == midx
            dq_k = jnp.sum(jnp.where(onehot, dq, 0.0), axis=1,
                           keepdims=True)
            dist_k = jnp.sqrt(jnp.clip(dq_k, 1e-12, None))
            dm = jnp.where(lane16 == k, jnp.broadcast_to(dist_k, (_B, 16)),
                           dm)
            dn = jnp.where(onehot, jnp.inf, dn)
        mx = jnp.max(dm, axis=1, keepdims=True)
        e = jnp.exp(dm - mx)
        w0 = e[:, 0:1] / jnp.sum(e, axis=1, keepdims=True)
        pred_ref[...] = (1.0 - w0) * maxsc_ref[...]


@jax.jit
def kernel(embedding, memory_bank):
    f32, i32 = jnp.float32, jnp.int32
    bank_t = memory_bank.T

    scores, q8, maxsc = pl.pallas_call(
        _stage1_body,
        grid=(_NR, _NC),
        in_specs=[
            pl.BlockSpec((_TR, _D), lambda i, j: (i, 0)),
            pl.BlockSpec((_D, _TC), lambda i, j: (0, j)),
        ],
        out_specs=[
            pl.BlockSpec((_TR, 1), lambda i, j: (i, 0)),
            pl.BlockSpec((_B, _D), lambda i, j: (0, 0)),
            pl.BlockSpec((_B, 1), lambda i, j: (0, 0)),
        ],
        out_shape=[
            jax.ShapeDtypeStruct((_B * _N, 1), f32),
            jax.ShapeDtypeStruct((_B, _D), f32),
            jax.ShapeDtypeStruct((_B, 1), f32),
        ],
        scratch_shapes=[pltpu.VMEM((_TR, 1), f32)],
        compiler_params=pltpu.CompilerParams(
            dimension_semantics=("arbitrary", "arbitrary")),
    )(embedding, bank_t)

    dq, nnidx = pl.pallas_call(
        _stage2a_body,
        grid=(_NC,),
        in_specs=[
            pl.BlockSpec((_B, _D), lambda j: (0, 0)),
            pl.BlockSpec((_D, _TC), lambda j: (0, j)),
        ],
        out_specs=[
            pl.BlockSpec((_B, _M), lambda j: (0, 0)),
            pl.BlockSpec((_B, 1), lambda j: (0, 0)),
        ],
        out_shape=[
            jax.ShapeDtypeStruct((_B, _M), f32),
            jax.ShapeDtypeStruct((_B, 1), i32),
        ],
        scratch_shapes=[
            pltpu.VMEM((_B, 1), f32),
            pltpu.VMEM((_B, 1), i32),
        ],
        compiler_params=pltpu.CompilerParams(
            dimension_semantics=("arbitrary",)),
    )(q8, bank_t)

    nn8 = pl.pallas_call(
        _gather_body,
        grid_spec=pltpu.PrefetchScalarGridSpec(
            num_scalar_prefetch=1,
            grid=(_B,),
            in_specs=[pl.BlockSpec((1, 1, _D), lambda b, idx: (idx[b], 0, 0))],
            out_specs=pl.BlockSpec((1, 1, _D), lambda b, idx: (b, 0, 0)),
        ),
        out_shape=jax.ShapeDtypeStruct((_B, 1, _D), f32),
    )(nnidx.reshape(_B), memory_bank.reshape(_M, 1, _D)).reshape(_B, _D)

    pred = pl.pallas_call(
        _stage2b_body,
        grid=(_NC,),
        in_specs=[
            pl.BlockSpec((_B, _D), lambda j: (0, 0)),
            pl.BlockSpec((_B, _M), lambda j: (0, 0)),
            pl.BlockSpec((_B, 1), lambda j: (0, 0)),
            pl.BlockSpec((_D, _TC), lambda j: (0, j)),
        ],
        out_specs=pl.BlockSpec((_B, 1), lambda j: (0, 0)),
        out_shape=jax.ShapeDtypeStruct((_B, 1), f32),
        scratch_shapes=[pltpu.VMEM((_B, _M), f32)],
        compiler_params=pltpu.CompilerParams(
            dimension_semantics=("arbitrary",)),
    )(nn8, dq, maxsc, bank_t)

    return scores.reshape(_B, _N), pred.reshape(_B)
